# baseline (speedup 1.0000x reference)
import functools
import jax
import jax.numpy as jnp
from jax import lax
from jax.experimental import pallas as pl
from jax.experimental.pallas import tpu as pltpu
from jax.experimental.pallas import tpu_sc as plsc

N = 10000
E = 160000
TPC = 16
EPT = E // TPC
RPT = 640
LAST = N - (TPC - 1) * RPT
NB = 5
BN = N // NB


def _make_sc_kernel(fo, K):
    ZC = 80
    assert EPT % K == 0 and K % 16 == 0 and K >= ZC
    assert RPT % ZC == 0 and LAST % ZC == 0
    mesh = plsc.VectorSubcoreMesh(core_axis_name="c", subcore_axis_name="s")
    out_type = ([jax.ShapeDtypeStruct((N, fo), jnp.float32)] * 6 +
                [jax.ShapeDtypeStruct((3 * N,), jnp.float32)] * 2)
    scratch = [
        pltpu.VMEM((6 * N,), jnp.float32),
        pltpu.VMEM((K, fo), jnp.float32),
        pltpu.VMEM((K,), jnp.int32),
        pltpu.VMEM((K,), jnp.int32),
        pltpu.VMEM((K,), jnp.float32),
        pltpu.VMEM((RPT,), jnp.float32),
        pltpu.VMEM_SHARED((N, fo), jnp.float32),
        pltpu.VMEM_SHARED((N,), jnp.float32),
        pltpu.SemaphoreType.DMA,
    ]

    @functools.partial(pl.kernel, mesh=mesh, out_type=out_type,
                       scratch_types=scratch,
                       compiler_params=pltpu.CompilerParams(
                           needs_layout_passes=False,
                           use_tc_tiling_on_sc=False))
    def sck(src_hbm, dst_hbm, h0, h1, h2, h3, h4, h5, esed_lo, esed_hi,
            o0, o1, o2, o3, o4, o5, den_lo, den_hi,
            esed_v, rows, srcb, dstb, eeb, zb, acc_sh, den_sh, sem):
        c = lax.axis_index("c")
        w = lax.axis_index("s")
        lanes = lax.iota(jnp.int32, 16)
        zv = jnp.zeros((16,), jnp.float32)
        dbase = w * RPT

        def run(hs, outs, esed_hbm, den_hbm):
            pltpu.sync_copy(esed_hbm, esed_v)

            def zzb(g, carry):
                zb[pl.ds(g * 16, 16)] = zv
                return carry
            lax.fori_loop(0, RPT // 16, zzb, 0)

            for j in range(3):
                def zrows(k, carry):
                    for t in range(fo // 16):
                        rows[k, pl.ds(t * 16, 16)] = zv
                    return carry
                lax.fori_loop(0, ZC, zrows, 0)

                @pl.when(w < TPC - 1)
                def _():
                    for m in range(RPT // ZC):
                        pltpu.sync_copy(rows.at[pl.ds(0, ZC)],
                                        acc_sh.at[pl.ds(dbase + m * ZC, ZC)])
                    pltpu.sync_copy(zb, den_sh.at[pl.ds(dbase, RPT)])

                @pl.when(w == TPC - 1)
                def _():
                    for m in range(LAST // ZC):
                        pltpu.sync_copy(rows.at[pl.ds(0, ZC)],
                                        acc_sh.at[pl.ds(dbase + m * ZC, ZC)])
                    pltpu.sync_copy(zb.at[pl.ds(0, LAST)],
                                    den_sh.at[pl.ds(dbase, LAST)])

                plsc.subcore_barrier()

                def chunk(i, carry):
                    eb = w * EPT + i * K
                    pltpu.sync_copy(src_hbm.at[pl.ds(eb, K)], srcb)
                    pltpu.sync_copy(dst_hbm.at[pl.ds(eb, K)], dstb)
                    cp = pltpu.async_copy(hs[j].at[srcb], rows, sem)

                    def grp(g, carry2):
                        s16 = srcb[pl.ds(g * 16, 16)]
                        t16 = dstb[pl.ds(g * 16, 16)]
                        es = plsc.load_gather(esed_v, [s16 * 6 + j])
                        ed = plsc.load_gather(esed_v, [t16 * 6 + (3 + j)])
                        e = es + ed
                        eeb[pl.ds(g * 16, 16)] = jnp.exp(
                            jnp.maximum(e, 0.2 * e))
                        return carry2
                    lax.fori_loop(0, K // 16, grp, 0)
                    cp.wait()

                    def sgrp(g, carry2):
                        k0 = g * 16
                        ridx = k0 + lanes
                        eev = eeb[pl.ds(k0, 16)]
                        for col in range(fo):
                            cidx = jnp.full((16,), col, jnp.int32)
                            v = plsc.load_gather(rows, [ridx, cidx])
                            plsc.store_scatter(rows, [ridx, cidx], v * eev)
                        return carry2
                    lax.fori_loop(0, K // 16, sgrp, 0)

                    pltpu.sync_copy(rows, acc_sh.at[dstb], add=True)
                    pltpu.sync_copy(eeb, den_sh.at[dstb], add=True)
                    return carry
                lax.fori_loop(0, EPT // K, chunk, 0)
                plsc.subcore_barrier()

                @pl.when(w < TPC - 1)
                def _():
                    pltpu.sync_copy(acc_sh.at[pl.ds(dbase, RPT)],
                                    outs[j].at[pl.ds(dbase, RPT)])
                    pltpu.sync_copy(den_sh.at[pl.ds(dbase, RPT)],
                                    den_hbm.at[pl.ds(j * N + dbase, RPT)])

                @pl.when(w == TPC - 1)
                def _():
                    pltpu.sync_copy(acc_sh.at[pl.ds(dbase, LAST)],
                                    outs[j].at[pl.ds(dbase, LAST)])
                    pltpu.sync_copy(den_sh.at[pl.ds(dbase, LAST)],
                                    den_hbm.at[pl.ds(j * N + dbase, LAST)])

        @pl.when(c == 0)
        def _():
            run((h0, h1, h2), (o0, o1, o2), esed_lo, den_lo)

        @pl.when(c == 1)
        def _():
            run((h3, h4, h5), (o3, o4, o5), esed_hi, den_hi)

    return sck


_SC_KERNELS = {16: _make_sc_kernel(16, 400),
               32: _make_sc_kernel(32, 400),
               64: _make_sc_kernel(64, 400)}


def _full(shape):
    return pl.BlockSpec(shape, lambda i: tuple(0 for _ in shape))


def _rows(shape):
    return pl.BlockSpec(shape, lambda i: (i,) + tuple(0 for _ in shape[1:]))


def _h_outs(fo):
    return ([_rows((BN, fo))] * 6 + [_rows((BN, 12))],
            [jax.ShapeDtypeStruct((N, fo), jnp.float32)] * 6 +
            [jax.ShapeDtypeStruct((N, 12), jnp.float32)])


def _split_heads(h, a, fo, refs):
    for j in range(6):
        refs[j][...] = h[:, j * fo:(j + 1) * fo]
    refs[6][...] = jnp.dot(h, a, preferred_element_type=jnp.float32)


def _tc_layer1(x, Wf, A):
    fi, W6 = Wf.shape
    fo = W6 // 6

    def body(x_ref, w_ref, a_ref, *o_refs):
        h = jnp.dot(x_ref[...], w_ref[...], preferred_element_type=jnp.float32)
        _split_heads(h, a_ref[...], fo, o_refs)

    out_specs, out_shape = _h_outs(fo)
    return pl.pallas_call(
        body,
        grid=(NB,),
        in_specs=[_rows((BN, fi)), _full((fi, W6)), _full((W6, 12))],
        out_specs=out_specs,
        out_shape=out_shape,
    )(x, Wf, A)


def _elu_x(accs, dn, fp):
    cols = [accs[j] / (dn[:, j:j + 1] + 1e-16) for j in range(6)]
    x = jnp.concatenate(cols, axis=1)
    return jnp.where(x > 0, x, jnp.exp(x) - 1.0)


def _tc_layer23(accs, den, Wf, A, fp):
    fi, W6 = Wf.shape
    fo = W6 // 6

    def body(a0, a1, a2, a3, a4, a5, dn_ref, w_ref, a_ref, *o_refs):
        x = _elu_x([a[...] for a in (a0, a1, a2, a3, a4, a5)], dn_ref[...], fp)
        h = jnp.dot(x, w_ref[...], preferred_element_type=jnp.float32)
        _split_heads(h, a_ref[...], fo, o_refs)

    out_specs, out_shape = _h_outs(fo)
    return pl.pallas_call(
        body,
        grid=(NB,),
        in_specs=[_rows((BN, fp))] * 6 + [_rows((BN, 6)),
                                          _full((fi, W6)), _full((W6, 12))],
        out_specs=out_specs,
        out_shape=out_shape,
    )(*accs, den, Wf, A)


def _tc_pool(acc_i, dni, acc_n, dnn):
    fp = 64

    def body(*refs):
        ai = [refs[j][...] for j in range(6)]
        dni_ref = refs[6]
        an = [refs[7 + j][...] for j in range(6)]
        dnn_ref = refs[13]
        o_ref = refs[14]
        xi = _elu_x(ai, dni_ref[...], fp)
        xn = _elu_x(an, dnn_ref[...], fp)
        s = jnp.concatenate([jnp.sum(xi, axis=0, keepdims=True),
                             jnp.sum(xn, axis=0, keepdims=True)], axis=1)

        @pl.when(pl.program_id(0) == 0)
        def _():
            o_ref[...] = s

        @pl.when(pl.program_id(0) != 0)
        def _():
            o_ref[...] = o_ref[...] + s

    return pl.pallas_call(
        body,
        grid=(NB,),
        in_specs=([_rows((BN, fp))] * 6 + [_rows((BN, 6))]) * 2,
        out_specs=_full((1, 768)),
        out_shape=jax.ShapeDtypeStruct((1, 768), jnp.float32),
    )(*acc_i, dni, *acc_n, dnn)


def _tc_head(s, Wd, bd):
    def body(s_ref, wd_ref, bd_ref, o_ref):
        sv = s_ref[...]
        s2 = jnp.sum(sv * sv, axis=1, keepdims=True)
        nrm = jnp.maximum(jnp.sqrt(s2), 1e-12)
        dot = jnp.dot(sv, wd_ref[...], preferred_element_type=jnp.float32)
        o_ref[...] = dot / nrm + bd_ref[...]

    return pl.pallas_call(
        body,
        out_shape=jax.ShapeDtypeStruct((1, 1), jnp.float32),
    )(s, Wd, bd)


def _prep(W, a_s, a_d):
    Hh, fi, fo = W.shape
    Wf = W.transpose(1, 0, 2).reshape(fi, Hh * fo)
    eye = jnp.eye(Hh, dtype=W.dtype)
    As = (eye[:, None, :] * a_s[:, :, None]).reshape(Hh * fo, Hh)
    Ad = (eye[:, None, :] * a_d[:, :, None]).reshape(Hh * fo, Hh)
    A = jnp.concatenate([As[:, :3], Ad[:, :3], As[:, 3:], Ad[:, 3:]], axis=1)
    return Wf, A


def _den6(den_lo, den_hi):
    return jnp.concatenate([den_lo.reshape(3, N),
                            den_hi.reshape(3, N)], axis=0).T


def _branch(x, src, dst, plist):
    (W1, as1, ad1), (W2, as2, ad2), (W3_, as3, ad3) = plist
    fos = [16, 32, 64]
    outs = _tc_layer1(x, *_prep(W1, as1, ad1))
    hs, esed = outs[:6], outs[6]
    res = _SC_KERNELS[16](src, dst, *hs,
                          esed[:, :6].reshape(-1), esed[:, 6:].reshape(-1))
    accs, den_lo, den_hi = res[:6], res[6], res[7]
    for l, (W, a_s, a_d) in ((2, (W2, as2, ad2)), (3, (W3_, as3, ad3))):
        den = _den6(den_lo, den_hi)
        outs = _tc_layer23(accs, den, *_prep(W, a_s, a_d), fp=fos[l - 2])
        hs, esed = outs[:6], outs[6]
        res = _SC_KERNELS[fos[l - 1]](src, dst, *hs,
                                      esed[:, :6].reshape(-1),
                                      esed[:, 6:].reshape(-1))
        accs, den_lo, den_hi = res[:6], res[6], res[7]
    return accs, _den6(den_lo, den_hi)


def kernel(x_int, x_nh, edge_index_int, edge_index_nh,
           W1i, as1i, ad1i, W1n, as1n, ad1n,
           W2i, as2i, ad2i, W2n, as2n, ad2n,
           W3i, as3i, ad3i, W3n, as3n, ad3n,
           Wd, bd):
    si, di = edge_index_int[0], edge_index_int[1]
    sn, dn = edge_index_nh[0], edge_index_nh[1]
    acc_i, dni = _branch(
        x_int, si, di,
        [(W1i, as1i, ad1i), (W2i, as2i, ad2i), (W3i, as3i, ad3i)])
    acc_n, dnn = _branch(
        x_nh, sn, dn,
        [(W1n, as1n, ad1n), (W2n, as2n, ad2n), (W3n, as3n, ad3n)])
    s = _tc_pool(acc_i, dni, acc_n, dnn)
    out = _tc_head(s, Wd, bd.reshape(1, 1))
    return out.reshape(1)
